# Initial kernel scaffold; baseline (speedup 1.0000x reference)
import jax
import jax.numpy as jnp
from jax.experimental import pallas as pl


def kernel(indices, W, hash_tables, p):
    raise NotImplementedError("write your pallas kernel here")


# speedup vs baseline: 1.9516x; 1.9516x over previous
import dataclasses
import functools

import jax
import jax.numpy as jnp
from jax import lax
from jax.experimental import pallas as pl
from jax.experimental.pallas import tpu as pltpu
from jax.experimental.pallas import tpu_sc as plsc

NC = 2
NS = 16
NW = NC * NS
LANES = 16
WT = 128
HTPK = 8


def _sc_body(htp_hbm, idx_hbm, w_hbm, out_hbm,
             idx_v, htp_v, b0_v, b1_v, p0_v, p1_v, w0_v, w1_v, out_v, sem,
             n_win, per_w, d):
    wid = lax.axis_index("s") * NC + lax.axis_index("c")
    iot = lax.iota(jnp.int32, LANES)

    @pl.loop(0, n_win)
    def _win(win):
        base = wid * per_w + win * WT

        pltpu.sync_copy(idx_hbm.at[pl.ds(base, WT)], idx_v)

        pltpu.async_copy(htp_hbm.at[idx_v], htp_v, sem).wait()

        for g in range(WT // LANES):
            tv = idx_v[pl.ds(g * LANES, LANES)]
            rowv = iot + (g * LANES)
            h0 = plsc.load_gather(htp_v, [rowv, jnp.full((LANES,), 0, jnp.int32)])
            h1 = plsc.load_gather(htp_v, [rowv, jnp.full((LANES,), 1, jnp.int32)])
            pb0 = plsc.load_gather(htp_v, [rowv, jnp.full((LANES,), 2, jnp.int32)])
            pb1 = plsc.load_gather(htp_v, [rowv, jnp.full((LANES,), 3, jnp.int32)])
            nz = tv != 0
            zero = jnp.zeros((LANES,), jnp.int32)
            b0_v[pl.ds(g * LANES, LANES)] = jnp.where(nz, h0, zero)
            b1_v[pl.ds(g * LANES, LANES)] = jnp.where(nz, h1, zero)
            p0_v[pl.ds(g * LANES, LANES)] = plsc.bitcast(pb0, jnp.float32)
            p1_v[pl.ds(g * LANES, LANES)] = plsc.bitcast(pb1, jnp.float32)

        pltpu.async_copy(w_hbm.at[b0_v], w0_v, sem).wait()
        pltpu.async_copy(w_hbm.at[b1_v], w1_v, sem).wait()

        @pl.loop(0, WT // LANES)
        def _grp(g):
            off = pl.multiple_of(g * LANES, LANES)
            p0vec = p0_v[pl.ds(off, LANES)]
            p1vec = p1_v[pl.ds(off, LANES)]
            rowv = iot + off
            for dd in range(d):
                colv = jnp.full((LANES,), dd, jnp.int32)
                w0c = plsc.load_gather(w0_v, [rowv, colv])
                w1c = plsc.load_gather(w1_v, [rowv, colv])
                plsc.store_scatter(out_v, [rowv, colv],
                                   w0c * p0vec + w1c * p1vec)

        pltpu.sync_copy(out_v, out_hbm.at[pl.ds(base, WT)])


def kernel(indices, W, hash_tables, p):
    b, l = indices.shape
    d = W.shape[1]
    n = b * l
    per_w = n // NW
    n_win = per_w // WT

    htp = jnp.concatenate(
        [hash_tables, lax.bitcast_convert_type(p, jnp.int32),
         jnp.zeros((hash_tables.shape[0], HTPK - 4), jnp.int32)], axis=1)
    idx1 = indices.reshape(n)

    mesh = plsc.VectorSubcoreMesh(core_axis_name="c", subcore_axis_name="s",
                                  num_cores=NC, num_subcores=NS)
    body = functools.partial(_sc_body, n_win=n_win, per_w=per_w, d=d)
    cp = pltpu.CompilerParams()
    for fld, val in (("needs_layout_passes", False),
                     ("use_tc_tiling_on_sc", False)):
        if fld in pltpu.CompilerParams.__dataclass_fields__:
            cp = dataclasses.replace(cp, **{fld: val})
    run = pl.kernel(
        body,
        out_type=jax.ShapeDtypeStruct((n, d), jnp.float32),
        mesh=mesh,
        compiler_params=cp,
        scratch_types=[
            pltpu.VMEM((WT,), jnp.int32),
            pltpu.VMEM((WT, HTPK), jnp.int32),
            pltpu.VMEM((WT,), jnp.int32),
            pltpu.VMEM((WT,), jnp.int32),
            pltpu.VMEM((WT,), jnp.float32),
            pltpu.VMEM((WT,), jnp.float32),
            pltpu.VMEM((WT, d), jnp.float32),
            pltpu.VMEM((WT, d), jnp.float32),
            pltpu.VMEM((WT, d), jnp.float32),
            pltpu.SemaphoreType.DMA,
        ],
    )
    out = run(htp, idx1, W)
    return out.reshape(b, l, d)
